# baseline (speedup 1.0000x reference)
import functools

import jax
import jax.numpy as jnp
from jax import lax
from jax.experimental import pallas as pl
from jax.experimental.pallas import tpu as pltpu
from jax.experimental.pallas import tpu_sc as plsc

NC, NS, L = 2, 16, 16
NW = NC * NS
C = 128
D = 128
BE = 1024


def _prep_body(x_ref, p_ref, w1_ref, w2_ref, b_ref, a_ref, bb_ref):
    pv = p_ref[...] @ w2_ref[...]
    a_ref[...] = x_ref[...] @ w1_ref[...] + pv + b_ref[...]
    bb_ref[...] = pv


def _dense_body(z_ref, wg_ref, bg_ref, msg_ref, gate_ref):
    z = z_ref[...]
    m = z * jax.nn.sigmoid(z)
    msg_ref[...] = m
    g = jnp.sum(m * wg_ref[...], axis=1, keepdims=True) + bg_ref[...]
    gate_ref[...] = g * jax.nn.sigmoid(g)


def _comb_body(p0_ref, p1_ref, o_ref):
    o_ref[...] = p0_ref[...] + p1_ref[...]


def _make_gather(Epad, Nacc):
    mesh = plsc.VectorSubcoreMesh(core_axis_name="c", subcore_axis_name="s",
                                  num_cores=NC, num_subcores=NS)
    f32 = jnp.float32

    @functools.partial(
        pl.kernel, mesh=mesh,
        out_type=jax.ShapeDtypeStruct((Epad, D), f32),
        scratch_types=[
            pltpu.VMEM((C,), jnp.int32), pltpu.VMEM((C,), jnp.int32),
            pltpu.VMEM((C,), jnp.int32), pltpu.VMEM((C,), jnp.int32),
            pltpu.VMEM((C, D), f32), pltpu.VMEM((C, D), f32),
            pltpu.VMEM((C, D), f32), pltpu.VMEM((C, D), f32),
            pltpu.SemaphoreType.DMA, pltpu.SemaphoreType.DMA,
            pltpu.SemaphoreType.DMA, pltpu.SemaphoreType.DMA,
        ],
    )
    def k(a_hbm, b_hbm, src_hbm, dst_hbm, z_hbm,
          sidx0, didx0, sidx1, didx1, a0, b0, a1, b1,
          sa0, sb0, sa1, sb1):
        wid = lax.axis_index("s") * NC + lax.axis_index("c")
        ept = Epad // NW
        base0 = wid * ept
        nch = ept // C

        def issue(base, sidx, didx, abuf, bbuf, sa, sb):
            pltpu.sync_copy(src_hbm.at[pl.ds(base, C)], sidx)
            pltpu.sync_copy(dst_hbm.at[pl.ds(base, C)], didx)
            ca = pltpu.async_copy(a_hbm.at[sidx], abuf, sa)
            cb = pltpu.async_copy(b_hbm.at[didx], bbuf, sb)
            return ca, cb

        def flush(base, abuf, bbuf, ca, cb):
            ca.wait()
            cb.wait()

            def ce(e, c):
                for j in range(D // L):
                    sl = pl.ds(j * L, L)
                    abuf[e, sl] = abuf[e, sl] - bbuf[e, sl]
                return c

            lax.fori_loop(0, C, ce, 0)
            pltpu.sync_copy(abuf, z_hbm.at[pl.ds(base, C)])

        def body(i, carry):
            e0 = base0 + 2 * i * C
            e1 = e0 + C
            c0 = issue(e0, sidx0, didx0, a0, b0, sa0, sb0)
            c1 = issue(e1, sidx1, didx1, a1, b1, sa1, sb1)
            flush(e0, a0, b0, *c0)
            flush(e1, a1, b1, *c1)
            return carry

        lax.fori_loop(0, nch // 2, body, 0)
        if nch % 2:
            tb = base0 + (nch - 1) * C
            ct = issue(tb, sidx0, didx0, a0, b0, sa0, sb0)
            flush(tb, a0, b0, *ct)

    return k


def _make_agg(Epad, Nacc):
    mesh = plsc.VectorSubcoreMesh(core_axis_name="c", subcore_axis_name="s",
                                  num_cores=NC, num_subcores=NS)
    f32 = jnp.float32
    SEG = Nacc // NS

    @functools.partial(
        pl.kernel, mesh=mesh,
        out_type=(jax.ShapeDtypeStruct((Nacc, D), f32),
                  jax.ShapeDtypeStruct((Nacc, D), f32)),
        scratch_types=[
            pltpu.VMEM((C,), jnp.int32),
            pltpu.VMEM((C,), f32),
            pltpu.VMEM((C,), f32),
            pltpu.VMEM((C,), f32),
            pltpu.VMEM((C, D), f32),
            pltpu.VMEM((1, L), f32),
            pltpu.VMEM((NS, L), f32),
            pltpu.VMEM_SHARED((Nacc,), f32),
            pltpu.VMEM_SHARED((Nacc, D), f32),
            pltpu.VMEM_SHARED((NS, L), f32),
            pltpu.SemaphoreType.DMA,
        ],
    )
    def k(gate_hbm, dst_hbm, msg_hbm, zrow_hbm, zacc_hbm, p0_hbm, p1_hbm,
          didx, gbuf, ebuf, dbuf, mrows, mx_v, mall_v,
          denom_sh, acc_sh, maxima_sh, sem):
        cid = lax.axis_index("c")
        sid = lax.axis_index("s")
        wid = sid * NC + cid
        ept16 = Epad // NS
        eptw = Epad // NW

        pltpu.sync_copy(zrow_hbm, denom_sh.at[pl.ds(sid * SEG, SEG)])
        pltpu.sync_copy(zacc_hbm, acc_sh.at[pl.ds(sid * SEG, SEG)])

        neg = jnp.full((L,), -1e30, f32)

        def amax_body(i, m):
            pltpu.sync_copy(gate_hbm.at[pl.ds(sid * ept16 + i * C, C)], gbuf)
            for j in range(C // L):
                m = jnp.maximum(m, gbuf[pl.ds(j * L, L)])
            return m

        m = lax.fori_loop(0, ept16 // C, amax_body, neg)
        mx_v[0, :] = m
        pltpu.sync_copy(mx_v, maxima_sh.at[pl.ds(sid, 1)])
        plsc.subcore_barrier()
        pltpu.sync_copy(maxima_sh, mall_v)
        gm = neg
        for s in range(NS):
            gm = jnp.maximum(gm, mall_v[s])
        lane = lax.iota(jnp.int32, L)
        for sh in (1, 2, 4, 8):
            gm = jnp.maximum(gm, gm[lane ^ sh])
        G = gm

        def db(i, carry):
            base = sid * ept16 + i * C
            pltpu.sync_copy(gate_hbm.at[pl.ds(base, C)], gbuf)
            pltpu.sync_copy(dst_hbm.at[pl.ds(base, C)], didx)
            for j in range(C // L):
                ebuf[pl.ds(j * L, L)] = jnp.exp(gbuf[pl.ds(j * L, L)] - G)
            pltpu.sync_copy(ebuf, denom_sh.at[didx], add=True)
            return carry

        lax.fori_loop(0, ept16 // C, db, 0)
        plsc.subcore_barrier()

        def wb(i, carry):
            base = wid * eptw + i * C
            pltpu.sync_copy(gate_hbm.at[pl.ds(base, C)], gbuf)
            pltpu.sync_copy(dst_hbm.at[pl.ds(base, C)], didx)
            pltpu.async_copy(msg_hbm.at[pl.ds(base, C)], mrows, sem).wait()
            pltpu.async_copy(denom_sh.at[didx], dbuf, sem).wait()
            for j in range(C // L):
                a = jnp.exp(gbuf[pl.ds(j * L, L)] - G) / (
                    dbuf[pl.ds(j * L, L)] + 1e-16)
                ebuf[pl.ds(j * L, L)] = a

            def rowb(g, carry2):
                av = ebuf[pl.ds(g * L, L)]
                for l in range(L):
                    bv = jnp.full((L,), av[l], f32)
                    e = g * L + l
                    for j in range(D // L):
                        mrows[e, pl.ds(j * L, L)] = (
                            mrows[e, pl.ds(j * L, L)] * bv)
                return carry2

            lax.fori_loop(0, C // L, rowb, 0)
            pltpu.sync_copy(mrows, acc_sh.at[didx], add=True)
            return carry

        lax.fori_loop(0, eptw // C, wb, 0)
        plsc.subcore_barrier()

        @pl.when(cid == 0)
        def _():
            pltpu.sync_copy(acc_sh.at[pl.ds(sid * SEG, SEG)],
                            p0_hbm.at[pl.ds(sid * SEG, SEG)])

        @pl.when(cid == 1)
        def _():
            pltpu.sync_copy(acc_sh.at[pl.ds(sid * SEG, SEG)],
                            p1_hbm.at[pl.ds(sid * SEG, SEG)])

    return k


def kernel(x, pos, W_local, b_local, W_gate, b_gate, edge_index):
    f32 = jnp.float32
    N = x.shape[0]
    E = edge_index.shape[1]

    loops = jnp.arange(N, dtype=edge_index.dtype)
    src = jnp.concatenate([edge_index[0], loops])
    dst = jnp.concatenate([edge_index[1], loops])
    Et = E + N
    Epad = ((Et + NW * C - 1) // (NW * C)) * (NW * C)
    Nacc = ((N + NS * 8 - 1) // (NS * 8)) * (NS * 8) + NS * 8
    pad_idx = N + 4
    pad = jnp.full((Epad - Et,), pad_idx, dtype=src.dtype)
    src = jnp.concatenate([src, pad])
    dst = jnp.concatenate([dst, pad])

    xp = jnp.zeros((Nacc, D), f32).at[:N].set(x)
    posP = jnp.zeros((Nacc, D), f32).at[:N, :3].set(pos)
    W1 = W_local[:D]
    W2 = jnp.zeros((D, D), f32).at[:3].set(W_local[D:])

    A, NB = pl.pallas_call(
        _prep_body,
        out_shape=(jax.ShapeDtypeStruct((Nacc, D), f32),
                   jax.ShapeDtypeStruct((Nacc, D), f32)),
    )(xp, posP, W1, W2, b_local.reshape(1, D))

    Z = _make_gather(Epad, Nacc)(A, NB, src, dst)

    nblk = Epad // BE
    msg, gcol = pl.pallas_call(
        _dense_body,
        grid=(nblk,),
        in_specs=[
            pl.BlockSpec((BE, D), lambda i: (i, 0)),
            pl.BlockSpec((1, D), lambda i: (0, 0)),
            pl.BlockSpec((1, 1), lambda i: (0, 0)),
        ],
        out_specs=[
            pl.BlockSpec((BE, D), lambda i: (i, 0)),
            pl.BlockSpec((BE, 1), lambda i: (i, 0)),
        ],
        out_shape=(jax.ShapeDtypeStruct((Epad, D), f32),
                   jax.ShapeDtypeStruct((Epad, 1), f32)),
    )(Z, W_gate.reshape(1, D), b_gate.reshape(1, 1))
    gate = gcol.reshape(Epad)

    SEG = Nacc // NS
    zrow = jnp.zeros((SEG,), f32)
    zacc = jnp.zeros((SEG, D), f32)
    P0, P1 = _make_agg(Epad, Nacc)(gate, dst, msg, zrow, zacc)

    NB5 = 2000
    out = pl.pallas_call(
        _comb_body,
        grid=(N // NB5,),
        in_specs=[pl.BlockSpec((NB5, D), lambda i: (i, 0)),
                  pl.BlockSpec((NB5, D), lambda i: (i, 0))],
        out_specs=pl.BlockSpec((NB5, D), lambda i: (i, 0)),
        out_shape=jax.ShapeDtypeStruct((N, D), f32),
    )(P0, P1)
    return out


# speedup vs baseline: 1.3291x; 1.3291x over previous
import functools

import jax
import jax.numpy as jnp
from jax import lax
from jax.experimental import pallas as pl
from jax.experimental.pallas import tpu as pltpu
from jax.experimental.pallas import tpu_sc as plsc

NC, NS, L = 2, 16, 16
NW = NC * NS
C = 128
D = 128
BE = 1024


def _prep_body(x_ref, p_ref, w1_ref, w2_ref, b_ref, a_ref, bb_ref):
    pv = p_ref[...] @ w2_ref[...]
    a_ref[...] = x_ref[...] @ w1_ref[...] + pv + b_ref[...]
    bb_ref[...] = pv


def _dense_body(z_ref, wg_ref, bg_ref, msg_ref, gate_ref):
    z = z_ref[...]
    m = z * jax.nn.sigmoid(z)
    msg_ref[...] = m
    g = jnp.sum(m * wg_ref[...], axis=1, keepdims=True) + bg_ref[...]
    gate_ref[...] = g * jax.nn.sigmoid(g)


def _comb_body(p0_ref, p1_ref, o_ref):
    o_ref[...] = p0_ref[...] + p1_ref[...]


def _make_gather(Epad, Nacc):
    mesh = plsc.VectorSubcoreMesh(core_axis_name="c", subcore_axis_name="s",
                                  num_cores=NC, num_subcores=NS)
    f32 = jnp.float32

    KC = C // 2
    @functools.partial(
        pl.kernel, mesh=mesh,
        out_type=jax.ShapeDtypeStruct((Epad, D), f32),
        scratch_types=[
            pltpu.VMEM((KC,), jnp.int32), pltpu.VMEM((KC,), jnp.int32),
            pltpu.VMEM((KC,), jnp.int32), pltpu.VMEM((KC,), jnp.int32),
            pltpu.VMEM((KC, D), f32), pltpu.VMEM((KC, D), f32),
            pltpu.VMEM((KC, D), f32), pltpu.VMEM((KC, D), f32),
            pltpu.VMEM_SHARED((Nacc, D), f32),
            pltpu.SemaphoreType.DMA, pltpu.SemaphoreType.DMA,
            pltpu.SemaphoreType.DMA, pltpu.SemaphoreType.DMA,
        ],
    )
    def k(a_hbm, b_hbm, src_hbm, dst_hbm, z_hbm,
          sidx0, didx0, sidx1, didx1, a0, b0, a1, b1, a_sh,
          sa0, sb0, sa1, sb1):
        sid = lax.axis_index("s")
        cid = lax.axis_index("c")
        wid = sid * NC + cid
        ept = Epad // NW
        base0 = wid * ept
        nch = ept // KC

        SEGK = Nacc // NS
        pltpu.sync_copy(a_hbm.at[pl.ds(sid * SEGK, SEGK)],
                        a_sh.at[pl.ds(sid * SEGK, SEGK)])
        plsc.subcore_barrier()

        def issue(base, sidx, didx, abuf, bbuf, sa, sb):
            pltpu.sync_copy(src_hbm.at[pl.ds(base, KC)], sidx)
            pltpu.sync_copy(dst_hbm.at[pl.ds(base, KC)], didx)
            ca = pltpu.async_copy(a_sh.at[sidx], abuf, sa)
            cb = pltpu.async_copy(b_hbm.at[didx], bbuf, sb)
            return ca, cb

        def flush(base, abuf, bbuf, ca, cb):
            ca.wait()
            cb.wait()

            def ce(e, c):
                for j in range(D // L):
                    sl = pl.ds(j * L, L)
                    abuf[e, sl] = abuf[e, sl] - bbuf[e, sl]
                return c

            lax.fori_loop(0, KC, ce, 0)
            pltpu.sync_copy(abuf, z_hbm.at[pl.ds(base, KC)])

        def body(i, carry):
            e0 = base0 + 2 * i * KC
            e1 = e0 + KC
            c0 = issue(e0, sidx0, didx0, a0, b0, sa0, sb0)
            c1 = issue(e1, sidx1, didx1, a1, b1, sa1, sb1)
            flush(e0, a0, b0, *c0)
            flush(e1, a1, b1, *c1)
            return carry

        lax.fori_loop(0, nch // 2, body, 0)
        if nch % 2:
            tb = base0 + (nch - 1) * KC
            ct = issue(tb, sidx0, didx0, a0, b0, sa0, sb0)
            flush(tb, a0, b0, *ct)

    return k


def _make_agg(Epad, Nacc):
    mesh = plsc.VectorSubcoreMesh(core_axis_name="c", subcore_axis_name="s",
                                  num_cores=NC, num_subcores=NS)
    f32 = jnp.float32
    SEG = Nacc // NS
    SUP = 16 * C

    @functools.partial(
        pl.kernel, mesh=mesh,
        out_type=(jax.ShapeDtypeStruct((Nacc, D), f32),
                  jax.ShapeDtypeStruct((Nacc, D), f32)),
        scratch_types=[
            pltpu.VMEM((SUP,), f32),
            tuple(pltpu.VMEM((C,), jnp.int32) for _ in range(SUP // C)),
            pltpu.VMEM((SUP // C, C), f32),
            pltpu.VMEM((C,), jnp.int32),
            pltpu.VMEM((C,), jnp.int32),
            pltpu.VMEM((C,), f32),
            pltpu.VMEM((C,), f32),
            pltpu.VMEM((C,), f32),
            pltpu.VMEM((C,), f32),
            pltpu.VMEM((C, D), f32),
            pltpu.VMEM((C, D), f32),
            pltpu.VMEM((1, L), f32),
            pltpu.VMEM((NS, L), f32),
            pltpu.VMEM_SHARED((Nacc,), f32),
            pltpu.VMEM_SHARED((Nacc, D), f32),
            pltpu.VMEM_SHARED((NS, L), f32),
            pltpu.SemaphoreType.DMA, pltpu.SemaphoreType.DMA,
            pltpu.SemaphoreType.DMA, pltpu.SemaphoreType.DMA,
        ],
    )
    def k(gate_hbm, dst_hbm, msg_hbm, zrow_hbm, zacc_hbm,
          p0_hbm, p1_hbm,
          gsup, dsupb, esup, didx0, didx1, gbuf0, gbuf1, dbuf0, dbuf1,
          mrows0, mrows1, mx_v, mall_v,
          denom_sh, acc_sh, maxima_sh, sm0, sm1, sd0, sd1):
        cid = lax.axis_index("c")
        sid = lax.axis_index("s")
        wid = sid * NC + cid
        ept16 = Epad // NS
        eptw = Epad // NW
        nsup = ept16 // SUP
        ntail = (ept16 % SUP) // C

        pltpu.sync_copy(zrow_hbm, denom_sh.at[pl.ds(sid * SEG, SEG)])
        pltpu.sync_copy(zacc_hbm, acc_sh.at[pl.ds(sid * SEG, SEG)])

        neg = jnp.full((L,), -1e30, f32)

        def amax_body(i, m):
            pltpu.sync_copy(gate_hbm.at[pl.ds(sid * ept16 + i * SUP, SUP)],
                            gsup)
            for j in range(SUP // L):
                m = jnp.maximum(m, gsup[pl.ds(j * L, L)])
            return m

        m = lax.fori_loop(0, nsup, amax_body, neg)
        if ntail:
            tbase = sid * ept16 + nsup * SUP
            pltpu.sync_copy(gate_hbm.at[pl.ds(tbase, ntail * C)],
                            gsup.at[pl.ds(0, ntail * C)])
            for j in range(ntail * C // L):
                m = jnp.maximum(m, gsup[pl.ds(j * L, L)])
        mx_v[0, :] = m
        pltpu.sync_copy(mx_v, maxima_sh.at[pl.ds(sid, 1)])
        plsc.subcore_barrier()
        pltpu.sync_copy(maxima_sh, mall_v)
        gm = neg
        for s in range(NS):
            gm = jnp.maximum(gm, mall_v[s])
        lane = lax.iota(jnp.int32, L)
        for sh in (1, 2, 4, 8):
            gm = jnp.maximum(gm, gm[lane ^ sh])
        G = gm

        SUPC = SUP // C

        def db_super(base, nck):
            cg = pltpu.async_copy(gate_hbm.at[pl.ds(base, nck * C)],
                                  gsup.at[pl.ds(0, nck * C)], sm0)
            cds = [pltpu.async_copy(dst_hbm.at[pl.ds(base + kk * C, C)],
                                    dsupb[kk], sm1)
                   for kk in range(nck)]
            cg.wait()
            for kk in range(nck):
                for j in range(C // L):
                    esup[kk, pl.ds(j * L, L)] = jnp.exp(
                        gsup[pl.ds(kk * C + j * L, L)] - G)
            for c in cds:
                c.wait()
            css = [pltpu.async_copy(esup.at[kk], denom_sh.at[dsupb[kk]],
                                    sd0, add=True)
                   for kk in range(nck)]
            for c in css:
                c.wait()

        def db(i, carry):
            db_super(sid * ept16 + i * SUP, SUPC)
            return carry

        lax.fori_loop(0, nsup, db, 0)
        if ntail:
            db_super(sid * ept16 + nsup * SUP, ntail)
        plsc.subcore_barrier()

        def issue_d(base, didx, gbuf, mrows, dbuf, sm, sd):
            pltpu.sync_copy(gate_hbm.at[pl.ds(base, C)], gbuf)
            pltpu.sync_copy(dst_hbm.at[pl.ds(base, C)], didx)
            cm = pltpu.async_copy(msg_hbm.at[pl.ds(base, C)], mrows, sm)
            cd = pltpu.async_copy(denom_sh.at[didx], dbuf, sd)
            return cm, cd

        def flush_d(didx, gbuf, mrows, dbuf, cm, cd):
            cm.wait()
            cd.wait()
            for j in range(C // L):
                sl = pl.ds(j * L, L)
                gbuf[sl] = jnp.exp(gbuf[sl] - G) / (dbuf[sl] + 1e-16)

            def rowb(g, carry2):
                av = gbuf[pl.ds(g * L, L)]
                for l in range(L):
                    bv = jnp.full((L,), av[l], f32)
                    e = g * L + l
                    for j in range(D // L):
                        mrows[e, pl.ds(j * L, L)] = (
                            mrows[e, pl.ds(j * L, L)] * bv)
                return carry2

            lax.fori_loop(0, C // L, rowb, 0)
            pltpu.sync_copy(mrows, acc_sh.at[didx], add=True)

        npair = eptw // C // 2
        dtail = eptw // C - 2 * npair

        def wb(i, carry):
            e0 = wid * eptw + 2 * i * C
            c0 = issue_d(e0, didx0, gbuf0, mrows0, dbuf0, sm0, sd0)
            c1 = issue_d(e0 + C, didx1, gbuf1, mrows1, dbuf1, sm1, sd1)
            flush_d(didx0, gbuf0, mrows0, dbuf0, *c0)
            flush_d(didx1, gbuf1, mrows1, dbuf1, *c1)
            return carry

        lax.fori_loop(0, npair, wb, 0)
        if dtail:
            tb = wid * eptw + 2 * npair * C
            ct = issue_d(tb, didx0, gbuf0, mrows0, dbuf0, sm0, sd0)
            flush_d(didx0, gbuf0, mrows0, dbuf0, *ct)
        plsc.subcore_barrier()

        @pl.when(cid == 0)
        def _():
            pltpu.sync_copy(acc_sh.at[pl.ds(sid * SEG, SEG)],
                            p0_hbm.at[pl.ds(sid * SEG, SEG)])

        @pl.when(cid == 1)
        def _():
            pltpu.sync_copy(acc_sh.at[pl.ds(sid * SEG, SEG)],
                            p1_hbm.at[pl.ds(sid * SEG, SEG)])

    return k


def kernel(x, pos, W_local, b_local, W_gate, b_gate, edge_index):
    f32 = jnp.float32
    N = x.shape[0]
    E = edge_index.shape[1]

    loops = jnp.arange(N, dtype=edge_index.dtype)
    src = jnp.concatenate([edge_index[0], loops])
    dst = jnp.concatenate([edge_index[1], loops])
    Et = E + N
    Epad = ((Et + NW * C - 1) // (NW * C)) * (NW * C)
    Nacc = ((N + NS * 8 - 1) // (NS * 8)) * (NS * 8) + NS * 8
    pad_idx = N + 4
    pad = jnp.full((Epad - Et,), pad_idx, dtype=src.dtype)
    src = jnp.concatenate([src, pad])
    dst = jnp.concatenate([dst, pad])

    xp = jnp.zeros((Nacc, D), f32).at[:N].set(x)
    posP = jnp.zeros((Nacc, D), f32).at[:N, :3].set(pos)
    W1 = W_local[:D]
    W2 = jnp.zeros((D, D), f32).at[:3].set(W_local[D:])

    A, NB = pl.pallas_call(
        _prep_body,
        out_shape=(jax.ShapeDtypeStruct((Nacc, D), f32),
                   jax.ShapeDtypeStruct((Nacc, D), f32)),
    )(xp, posP, W1, W2, b_local.reshape(1, D))

    Z = _make_gather(Epad, Nacc)(A, NB, src, dst)

    nblk = Epad // BE
    msg, gcol = pl.pallas_call(
        _dense_body,
        grid=(nblk,),
        in_specs=[
            pl.BlockSpec((BE, D), lambda i: (i, 0)),
            pl.BlockSpec((1, D), lambda i: (0, 0)),
            pl.BlockSpec((1, 1), lambda i: (0, 0)),
        ],
        out_specs=[
            pl.BlockSpec((BE, D), lambda i: (i, 0)),
            pl.BlockSpec((BE, 1), lambda i: (i, 0)),
        ],
        out_shape=(jax.ShapeDtypeStruct((Epad, D), f32),
                   jax.ShapeDtypeStruct((Epad, 1), f32)),
    )(Z, W_gate.reshape(1, D), b_gate.reshape(1, 1))
    gate = gcol.reshape(Epad)

    SEG = Nacc // NS
    zrow = jnp.zeros((SEG,), f32)
    zacc = jnp.zeros((SEG, D), f32)
    P0, P1 = _make_agg(Epad, Nacc)(gate, dst, msg, zrow, zacc)

    NB5 = 2000
    out = pl.pallas_call(
        _comb_body,
        grid=(N // NB5,),
        in_specs=[pl.BlockSpec((NB5, D), lambda i: (i, 0)),
                  pl.BlockSpec((NB5, D), lambda i: (i, 0))],
        out_specs=pl.BlockSpec((NB5, D), lambda i: (i, 0)),
        out_shape=jax.ShapeDtypeStruct((N, D), f32),
    )(P0, P1)
    return out


# speedup vs baseline: 1.5167x; 1.1411x over previous
import functools

import jax
import jax.numpy as jnp
from jax import lax
from jax.experimental import pallas as pl
from jax.experimental.pallas import tpu as pltpu
from jax.experimental.pallas import tpu_sc as plsc

NC, NS, L = 2, 16, 16
NW = NC * NS
C = 128
D = 128
BE = 1024


def _prep_body(x_ref, p_ref, w1_ref, w2_ref, b_ref, a_ref):
    a_ref[...] = (x_ref[...] @ w1_ref[...] + p_ref[...] @ w2_ref[...]
                  + b_ref[...])


def _dense_body(z_ref, wg_ref, bg_ref, msg_ref, gate_ref):
    z = z_ref[...]
    m = z * jax.nn.sigmoid(z)
    msg_ref[...] = m
    g = jnp.sum(m * wg_ref[...], axis=1, keepdims=True) + bg_ref[...]
    gate_ref[...] = g * jax.nn.sigmoid(g)


def _comb_body(p0_ref, p1_ref, o_ref):
    o_ref[...] = p0_ref[...] + p1_ref[...]


def _make_gather(Epad, Nacc):
    mesh = plsc.VectorSubcoreMesh(core_axis_name="c", subcore_axis_name="s",
                                  num_cores=NC, num_subcores=NS)
    f32 = jnp.float32

    PW = 16

    @functools.partial(
        pl.kernel, mesh=mesh,
        out_type=jax.ShapeDtypeStruct((Epad, D), f32),
        scratch_types=[
            pltpu.VMEM((C,), jnp.int32), pltpu.VMEM((C,), jnp.int32),
            pltpu.VMEM((C,), jnp.int32), pltpu.VMEM((C,), jnp.int32),
            pltpu.VMEM((C, D), f32), tuple(pltpu.VMEM((C,), f32)
                                           for _ in range(3)),
            pltpu.VMEM((C, D), f32), tuple(pltpu.VMEM((C,), f32)
                                           for _ in range(3)),
            pltpu.VMEM((8, D), f32),
            pltpu.VMEM_SHARED((Nacc, D), f32),
            pltpu.SemaphoreType.DMA, pltpu.SemaphoreType.DMA,
            pltpu.SemaphoreType.DMA, pltpu.SemaphoreType.DMA,
        ],
    )
    def k(a_hbm, px_hbm, py_hbm, pz_hbm, w2_hbm, src_hbm, dst_hbm, z_hbm,
          sidx0, didx0, sidx1, didx1, a0, p0, a1, p1, w2v, a_sh,
          sa0, sb0, sa1, sb1):
        pcomp_hbm = (px_hbm, py_hbm, pz_hbm)
        sid = lax.axis_index("s")
        cid = lax.axis_index("c")
        wid = sid * NC + cid
        ept = Epad // NW
        base0 = wid * ept
        nch = ept // C

        pltpu.sync_copy(w2_hbm, w2v)
        w2r = [[w2v[kk, pl.ds(j * L, L)] for j in range(D // L)]
               for kk in range(3)]

        SEGK = Nacc // NS
        pltpu.sync_copy(a_hbm.at[pl.ds(sid * SEGK, SEGK)],
                        a_sh.at[pl.ds(sid * SEGK, SEGK)])
        plsc.subcore_barrier()

        def issue(base, sidx, didx, abuf, pbuf, sa, sb):
            pltpu.sync_copy(src_hbm.at[pl.ds(base, C)], sidx)
            pltpu.sync_copy(dst_hbm.at[pl.ds(base, C)], didx)
            ca = pltpu.async_copy(a_sh.at[sidx], abuf, sa)
            cbs = [pltpu.async_copy(pcomp_hbm[kk].at[didx], pbuf[kk], sb)
                   for kk in range(3)]
            return ca, cbs

        def flush(base, abuf, pbuf, ca, cbs):
            ca.wait()
            for cb in cbs:
                cb.wait()

            def ce(g, c):
                pxv = pbuf[0][pl.ds(g * L, L)]
                pyv = pbuf[1][pl.ds(g * L, L)]
                pzv = pbuf[2][pl.ds(g * L, L)]
                for l in range(L):
                    b0 = jnp.full((L,), pxv[l], f32)
                    b1 = jnp.full((L,), pyv[l], f32)
                    b2 = jnp.full((L,), pzv[l], f32)
                    e = g * L + l
                    for j in range(D // L):
                        sl = pl.ds(j * L, L)
                        abuf[e, sl] = (abuf[e, sl] - b0 * w2r[0][j]
                                       - b1 * w2r[1][j] - b2 * w2r[2][j])
                return c

            lax.fori_loop(0, C // L, ce, 0)
            pltpu.sync_copy(abuf, z_hbm.at[pl.ds(base, C)])

        def body(i, carry):
            e0 = base0 + 2 * i * C
            e1 = e0 + C
            c0 = issue(e0, sidx0, didx0, a0, p0, sa0, sb0)
            c1 = issue(e1, sidx1, didx1, a1, p1, sa1, sb1)
            flush(e0, a0, p0, *c0)
            flush(e1, a1, p1, *c1)
            return carry

        lax.fori_loop(0, nch // 2, body, 0)
        if nch % 2:
            tb = base0 + (nch - 1) * C
            ct = issue(tb, sidx0, didx0, a0, p0, sa0, sb0)
            flush(tb, a0, p0, *ct)

    return k


def _make_agg(Epad, Nacc):
    mesh = plsc.VectorSubcoreMesh(core_axis_name="c", subcore_axis_name="s",
                                  num_cores=NC, num_subcores=NS)
    f32 = jnp.float32
    SEG = Nacc // NS
    SUP = 16 * C

    @functools.partial(
        pl.kernel, mesh=mesh,
        out_type=(jax.ShapeDtypeStruct((Nacc, D), f32),
                  jax.ShapeDtypeStruct((Nacc, D), f32)),
        scratch_types=[
            pltpu.VMEM((SUP,), f32),
            tuple(pltpu.VMEM((C,), jnp.int32) for _ in range(SUP // C)),
            pltpu.VMEM((SUP // C, C), f32),
            pltpu.VMEM((C,), jnp.int32),
            pltpu.VMEM((C,), jnp.int32),
            pltpu.VMEM((C,), f32),
            pltpu.VMEM((C,), f32),
            pltpu.VMEM((C,), f32),
            pltpu.VMEM((C,), f32),
            pltpu.VMEM((C, D), f32),
            pltpu.VMEM((C, D), f32),
            pltpu.VMEM((1, L), f32),
            pltpu.VMEM((NS, L), f32),
            pltpu.VMEM_SHARED((Nacc,), f32),
            pltpu.VMEM_SHARED((Nacc, D), f32),
            pltpu.VMEM_SHARED((NS, L), f32),
            pltpu.SemaphoreType.DMA, pltpu.SemaphoreType.DMA,
            pltpu.SemaphoreType.DMA, pltpu.SemaphoreType.DMA,
        ],
    )
    def k(gate_hbm, dst_hbm, msg_hbm, zrow_hbm, zacc_hbm,
          p0_hbm, p1_hbm,
          gsup, dsupb, esup, didx0, didx1, gbuf0, gbuf1, dbuf0, dbuf1,
          mrows0, mrows1, mx_v, mall_v,
          denom_sh, acc_sh, maxima_sh, sm0, sm1, sd0, sd1):
        cid = lax.axis_index("c")
        sid = lax.axis_index("s")
        wid = sid * NC + cid
        ept16 = Epad // NS
        eptw = Epad // NW
        nsup = ept16 // SUP
        ntail = (ept16 % SUP) // C

        pltpu.sync_copy(zrow_hbm, denom_sh.at[pl.ds(sid * SEG, SEG)])
        pltpu.sync_copy(zacc_hbm, acc_sh.at[pl.ds(sid * SEG, SEG)])

        neg = jnp.full((L,), -1e30, f32)

        def amax_body(i, m):
            pltpu.sync_copy(gate_hbm.at[pl.ds(sid * ept16 + i * SUP, SUP)],
                            gsup)
            for j in range(SUP // L):
                m = jnp.maximum(m, gsup[pl.ds(j * L, L)])
            return m

        m = lax.fori_loop(0, nsup, amax_body, neg)
        if ntail:
            tbase = sid * ept16 + nsup * SUP
            pltpu.sync_copy(gate_hbm.at[pl.ds(tbase, ntail * C)],
                            gsup.at[pl.ds(0, ntail * C)])
            for j in range(ntail * C // L):
                m = jnp.maximum(m, gsup[pl.ds(j * L, L)])
        mx_v[0, :] = m
        pltpu.sync_copy(mx_v, maxima_sh.at[pl.ds(sid, 1)])
        plsc.subcore_barrier()
        pltpu.sync_copy(maxima_sh, mall_v)
        gm = neg
        for s in range(NS):
            gm = jnp.maximum(gm, mall_v[s])
        lane = lax.iota(jnp.int32, L)
        for sh in (1, 2, 4, 8):
            gm = jnp.maximum(gm, gm[lane ^ sh])
        G = gm

        SUPC = SUP // C

        def db_super(base, nck):
            cg = pltpu.async_copy(gate_hbm.at[pl.ds(base, nck * C)],
                                  gsup.at[pl.ds(0, nck * C)], sm0)
            cds = [pltpu.async_copy(dst_hbm.at[pl.ds(base + kk * C, C)],
                                    dsupb[kk], sm1)
                   for kk in range(nck)]
            cg.wait()
            for kk in range(nck):
                for j in range(C // L):
                    esup[kk, pl.ds(j * L, L)] = jnp.exp(
                        gsup[pl.ds(kk * C + j * L, L)] - G)
            for c in cds:
                c.wait()
            css = [pltpu.async_copy(esup.at[kk], denom_sh.at[dsupb[kk]],
                                    sd0, add=True)
                   for kk in range(nck)]
            for c in css:
                c.wait()

        def db(i, carry):
            db_super(sid * ept16 + i * SUP, SUPC)
            return carry

        lax.fori_loop(0, nsup, db, 0)
        if ntail:
            db_super(sid * ept16 + nsup * SUP, ntail)
        plsc.subcore_barrier()

        def issue_d(base, didx, gbuf, mrows, dbuf, sm, sd):
            pltpu.sync_copy(gate_hbm.at[pl.ds(base, C)], gbuf)
            pltpu.sync_copy(dst_hbm.at[pl.ds(base, C)], didx)
            cm = pltpu.async_copy(msg_hbm.at[pl.ds(base, C)], mrows, sm)
            cd = pltpu.async_copy(denom_sh.at[didx], dbuf, sd)
            return cm, cd

        def flush_d(didx, gbuf, mrows, dbuf, cm, cd):
            cm.wait()
            cd.wait()
            for j in range(C // L):
                sl = pl.ds(j * L, L)
                gbuf[sl] = jnp.exp(gbuf[sl] - G) / (dbuf[sl] + 1e-16)

            def rowb(g, carry2):
                av = gbuf[pl.ds(g * L, L)]
                for l in range(L):
                    bv = jnp.full((L,), av[l], f32)
                    e = g * L + l
                    for j in range(D // L):
                        mrows[e, pl.ds(j * L, L)] = (
                            mrows[e, pl.ds(j * L, L)] * bv)
                return carry2

            lax.fori_loop(0, C // L, rowb, 0)
            pltpu.sync_copy(mrows, acc_sh.at[didx], add=True)

        npair = eptw // C // 2
        dtail = eptw // C - 2 * npair

        def wb(i, carry):
            e0 = wid * eptw + 2 * i * C
            c0 = issue_d(e0, didx0, gbuf0, mrows0, dbuf0, sm0, sd0)
            c1 = issue_d(e0 + C, didx1, gbuf1, mrows1, dbuf1, sm1, sd1)
            flush_d(didx0, gbuf0, mrows0, dbuf0, *c0)
            flush_d(didx1, gbuf1, mrows1, dbuf1, *c1)
            return carry

        lax.fori_loop(0, npair, wb, 0)
        if dtail:
            tb = wid * eptw + 2 * npair * C
            ct = issue_d(tb, didx0, gbuf0, mrows0, dbuf0, sm0, sd0)
            flush_d(didx0, gbuf0, mrows0, dbuf0, *ct)
        plsc.subcore_barrier()

        @pl.when(cid == 0)
        def _():
            pltpu.sync_copy(acc_sh.at[pl.ds(sid * SEG, SEG)],
                            p0_hbm.at[pl.ds(sid * SEG, SEG)])

        @pl.when(cid == 1)
        def _():
            pltpu.sync_copy(acc_sh.at[pl.ds(sid * SEG, SEG)],
                            p1_hbm.at[pl.ds(sid * SEG, SEG)])

    return k


def kernel(x, pos, W_local, b_local, W_gate, b_gate, edge_index):
    f32 = jnp.float32
    N = x.shape[0]
    E = edge_index.shape[1]

    loops = jnp.arange(N, dtype=edge_index.dtype)
    src = jnp.concatenate([edge_index[0], loops])
    dst = jnp.concatenate([edge_index[1], loops])
    Et = E + N
    Epad = ((Et + NW * C - 1) // (NW * C)) * (NW * C)
    Nacc = ((N + NS * 8 - 1) // (NS * 8)) * (NS * 8) + NS * 8
    pad_idx = N + 4
    pad = jnp.full((Epad - Et,), pad_idx, dtype=src.dtype)
    src = jnp.concatenate([src, pad])
    dst = jnp.concatenate([dst, pad])

    xp = jnp.zeros((Nacc, D), f32).at[:N].set(x)
    posP = jnp.zeros((Nacc, D), f32).at[:N, :3].set(pos)
    W1 = W_local[:D]
    W2 = jnp.zeros((D, D), f32).at[:3].set(W_local[D:])

    A = pl.pallas_call(
        _prep_body,
        out_shape=jax.ShapeDtypeStruct((Nacc, D), f32),
    )(xp, posP, W1, W2, b_local.reshape(1, D))

    posc = jnp.zeros((Nacc, 3), f32).at[:N].set(pos)
    W2p = jnp.zeros((8, D), f32).at[:3].set(W_local[D:])
    Z = _make_gather(Epad, Nacc)(A, posc[:, 0], posc[:, 1], posc[:, 2],
                                 W2p, src, dst)

    nblk = Epad // BE
    msg, gcol = pl.pallas_call(
        _dense_body,
        grid=(nblk,),
        in_specs=[
            pl.BlockSpec((BE, D), lambda i: (i, 0)),
            pl.BlockSpec((1, D), lambda i: (0, 0)),
            pl.BlockSpec((1, 1), lambda i: (0, 0)),
        ],
        out_specs=[
            pl.BlockSpec((BE, D), lambda i: (i, 0)),
            pl.BlockSpec((BE, 1), lambda i: (i, 0)),
        ],
        out_shape=(jax.ShapeDtypeStruct((Epad, D), f32),
                   jax.ShapeDtypeStruct((Epad, 1), f32)),
    )(Z, W_gate.reshape(1, D), b_gate.reshape(1, 1))
    gate = gcol.reshape(Epad)

    SEG = Nacc // NS
    zrow = jnp.zeros((SEG,), f32)
    zacc = jnp.zeros((SEG, D), f32)
    P0, P1 = _make_agg(Epad, Nacc)(gate, dst, msg, zrow, zacc)

    NB5 = 2000
    out = pl.pallas_call(
        _comb_body,
        grid=(N // NB5,),
        in_specs=[pl.BlockSpec((NB5, D), lambda i: (i, 0)),
                  pl.BlockSpec((NB5, D), lambda i: (i, 0))],
        out_specs=pl.BlockSpec((NB5, D), lambda i: (i, 0)),
        out_shape=jax.ShapeDtypeStruct((N, D), f32),
    )(P0, P1)
    return out
